# baseline (speedup 1.0000x reference)
import jax
import jax.numpy as jnp
from jax import lax
from jax.experimental import pallas as pl
from jax.experimental.pallas import tpu as pltpu
from jax.experimental.pallas import tpu_sc as plsc

B, C, H, W = 8, 128, 128, 128
HW = H * W
M = B * HW
CB = 8
NW = 32
CHUNK = M // NW
PAIRS = CHUNK // W


def _tc_body(x_ref, xs_ref):
    cb = pl.program_id(1)
    partial = jnp.sum(jnp.abs(x_ref[0]), axis=0)

    @pl.when(cb == 0)
    def _():
        xs_ref[0, 0, :] = partial

    @pl.when(cb != 0)
    def _():
        xs_ref[0, 0, :] += partial


def _tc_abssum(x):
    xr = x.reshape(B, C, HW)
    xs = pl.pallas_call(
        _tc_body,
        grid=(B, CB),
        in_specs=[pl.BlockSpec((1, C // CB, HW), lambda b, cb: (b, cb, 0))],
        out_specs=pl.BlockSpec((1, 1, HW), lambda b, cb: (b, 0, 0)),
        out_shape=jax.ShapeDtypeStruct((B, 1, HW), jnp.float32),
    )(xr)
    return xs.reshape(M)


def _sc_body(xs_hbm, xrows_hbm, feats_hbm, crd_hbm,
             xs_c, crd_v, dest_v, idx_v, cnt_stage, cnt_v,
             cbuf0, cbuf1, tbuf0, tbuf1,
             cnt_sh, gs0, gs1, ss0, ss1):
    cbufs = (cbuf0, cbuf1)
    tbufs = (tbuf0, tbuf1)
    gsems = (gs0, gs1)
    ssems = (ss0, ss1)

    c = lax.axis_index("c")
    s = lax.axis_index("s")
    w = c * 16 + s
    w2 = (1 - c) * 16 + s
    o = w * CHUNK

    iota = lax.iota(jnp.int32, 16)
    zero16 = jnp.zeros((16,), jnp.int32)

    def _count(chunk_id):
        pltpu.sync_copy(xs_hbm.at[pl.ds(chunk_id * CHUNK, CHUNK)], xs_c)

        def body(i, acc):
            v = xs_c[pl.ds(i * 16, 16)]
            ai = jnp.where(v != 0.0, 1, 0).astype(jnp.int32)
            return acc + jnp.sum(ai, axis=0)

        return lax.fori_loop(0, CHUNK // 16, body, jnp.int32(0))

    n1 = _count(w)
    n2 = _count(w2)

    for kk in range(8):
        cnt_stage[pl.ds(kk * 16, 16)] = jnp.full((16,), n1, jnp.int32)
    pltpu.sync_copy(cnt_stage, cnt_sh.at[w])
    for kk in range(8):
        cnt_stage[pl.ds(kk * 16, 16)] = jnp.full((16,), n2, jnp.int32)
    pltpu.sync_copy(cnt_stage, cnt_sh.at[w2])
    plsc.subcore_barrier()
    pltpu.sync_copy(cnt_sh, cnt_v)

    lo = plsc.load_gather(cnt_v, [iota, zero16])
    hi = plsc.load_gather(cnt_v, [iota + 16, zero16])
    sum_lo = jnp.sum(lo, axis=0)
    tot_a = sum_lo + jnp.sum(hi, axis=0)
    ex_lo = plsc.cumsum(lo) - lo
    ex_hi = plsc.cumsum(hi) - hi + sum_lo

    def _process_chunk(w_, n_w, base_a):
        ib = tot_a + (w_ * CHUNK - base_a)
        ov_a = jnp.logical_and(base_a < o + CHUNK, base_a + n_w > o)
        ov_i = jnp.logical_and(ib < o + CHUNK, ib + (CHUNK - n_w) > o)

        @pl.when(jnp.logical_or(ov_a, ov_i))
        def _():
            pltpu.sync_copy(xs_hbm.at[pl.ds(w_ * CHUNK, CHUNK)], xs_c)

            def stream(i, carry):
                v = xs_c[pl.ds(i * 16, 16)]
                act = v != 0.0
                ai = jnp.where(act, 1, 0).astype(jnp.int32)
                excl = plsc.cumsum(ai) - ai + carry
                li = i * 16 + iota
                out = jnp.where(act, base_a + excl, ib + (li - excl)) - o
                msk = jnp.logical_and(out >= 0, out < CHUNK)
                idx = jnp.minimum(jnp.maximum(out, 0), CHUNK - 1)
                g = w_ * CHUNK + li
                bc = lax.shift_right_logical(g, 14)
                hc = jnp.bitwise_and(lax.shift_right_logical(g, 7), 127)
                wc = jnp.bitwise_and(g, 127)
                idx3 = idx * 3
                plsc.store_scatter(crd_v, [idx3], bc, mask=msk)
                plsc.store_scatter(crd_v, [idx3 + 1], hc, mask=msk)
                plsc.store_scatter(crd_v, [idx3 + 2], wc, mask=msk)
                return carry + jnp.sum(ai, axis=0)

            lax.fori_loop(0, CHUNK // 16, stream, jnp.int32(0))

    for w_ in range(NW):
        if w_ < 16:
            _process_chunk(w_, lo[w_], ex_lo[w_])
        else:
            _process_chunk(w_, hi[w_ - 16], ex_hi[w_ - 16])

    pltpu.sync_copy(crd_v, crd_hbm.at[pl.ds(o * 3, CHUNK * 3)])

    my_base_a = (jnp.sum(jnp.where(iota < w, lo, 0), axis=0)
                 + jnp.sum(jnp.where(iota + 16 < w, hi, 0), axis=0))
    my_ib = tot_a + (o - my_base_a)
    pltpu.sync_copy(xs_hbm.at[pl.ds(o, CHUNK)], xs_c)

    def dchunk(p, carry):
        g0 = o + p * 128
        bp = lax.shift_right_logical(g0, 14) * (C * H) + \
            jnp.bitwise_and(lax.shift_right_logical(g0, 7), 127)

        def sub(k, kc):
            off = p * 128 + k * 16
            v = xs_c[pl.ds(off, 16)]
            act = v != 0.0
            ai = jnp.where(act, 1, 0).astype(jnp.int32)
            excl = plsc.cumsum(ai) - ai + kc
            li = off + iota
            d = jnp.where(act, my_base_a + excl, my_ib + (li - excl))
            dest_v[p, pl.ds(k * 16, 16)] = d
            idx_v[p, pl.ds(k * 16, 16)] = bp + (k * 16 + iota) * H
            return kc + jnp.sum(ai, axis=0)

        return lax.fori_loop(0, 8, sub, carry)

    lax.fori_loop(0, PAIRS, dchunk, jnp.int32(0))

    def _g(p, b):
        return pltpu.async_copy(xrows_hbm.at[idx_v.at[p]], cbufs[b], gsems[b])

    def _s(p, b):
        return pltpu.async_copy(tbufs[b], feats_hbm.at[dest_v.at[p]],
                                ssems[b])

    def _transpose(b):
        cbuf = cbufs[b]
        tbuf = tbufs[b]

        def trow(r, _):
            rr = jnp.full((16,), r, jnp.int32)
            for k in range(8):
                v = plsc.load_gather(cbuf, [iota + k * 16, rr])
                tbuf[r, pl.ds(k * 16, 16)] = v
            return 0

        lax.fori_loop(0, W, trow, 0)

    gd = {}
    sd = {}
    gd[0] = _g(0, 0)
    gd[1] = _g(1, 1)
    for p in range(PAIRS):
        b = p & 1
        gd[p].wait()
        if p >= 2:
            sd[p - 2].wait()
        _transpose(b)
        if p + 2 < PAIRS:
            gd[p + 2] = _g(p + 2, b)
        sd[p] = _s(p, b)
    sd[PAIRS - 2].wait()
    sd[PAIRS - 1].wait()


def _sc_compact(xs, xrows):
    mesh = plsc.VectorSubcoreMesh(core_axis_name="c", subcore_axis_name="s",
                                  num_cores=2, num_subcores=16)
    kern = pl.kernel(
        _sc_body,
        out_type=[
            jax.ShapeDtypeStruct((M, C), jnp.float32),
            jax.ShapeDtypeStruct((M * 3,), jnp.int32),
        ],
        mesh=mesh,
        scratch_types=[
            pltpu.VMEM((CHUNK,), jnp.float32),
            pltpu.VMEM((CHUNK * 3,), jnp.int32),
            pltpu.VMEM((PAIRS, 128), jnp.int32),
            pltpu.VMEM((PAIRS, 128), jnp.int32),
            pltpu.VMEM((128,), jnp.int32),
            pltpu.VMEM((NW, 128), jnp.int32),
            pltpu.VMEM((C, W), jnp.float32),
            pltpu.VMEM((C, W), jnp.float32),
            pltpu.VMEM((W, C), jnp.float32),
            pltpu.VMEM((W, C), jnp.float32),
            pltpu.VMEM_SHARED((NW, 128), jnp.int32),
            pltpu.SemaphoreType.DMA,
            pltpu.SemaphoreType.DMA,
            pltpu.SemaphoreType.DMA,
            pltpu.SemaphoreType.DMA,
        ],
        compiler_params=pltpu.CompilerParams(needs_layout_passes=False),
    )
    return kern(xs, xrows)


@jax.jit
def kernel(x):
    xs = _tc_abssum(x)
    feats, coords_flat = _sc_compact(xs, x.reshape(B * C * H, W))
    return coords_flat.reshape(M, 3), feats


# speedup vs baseline: 2.2327x; 2.2327x over previous
import jax
import jax.numpy as jnp
from jax import lax
from jax.experimental import pallas as pl
from jax.experimental.pallas import tpu as pltpu
from jax.experimental.pallas import tpu_sc as plsc

B, C, H, W = 8, 128, 128, 128
HW = H * W
M = B * HW
NBLK = 2
BLK = HW // NBLK
NW = 32
CHUNK = M // NW
SLABS = CHUNK // 128


def _tc_body(x_ref, xt_ref, xs_ref):
    xin = x_ref[0]
    xt_ref[...] = xin.T
    xs_ref[0, 0, :] = jnp.sum(jnp.abs(xin), axis=0)


def _tc_transpose(x):
    xr = x.reshape(B, C, HW)
    xt, xs = pl.pallas_call(
        _tc_body,
        grid=(B, NBLK),
        in_specs=[pl.BlockSpec((1, C, BLK), lambda b, s: (b, 0, s))],
        out_specs=[
            pl.BlockSpec((BLK, C), lambda b, s: (b * NBLK + s, 0)),
            pl.BlockSpec((1, 1, BLK), lambda b, s: (b * NBLK + s, 0, 0)),
        ],
        out_shape=[
            jax.ShapeDtypeStruct((M, C), jnp.float32),
            jax.ShapeDtypeStruct((B * NBLK, 1, BLK), jnp.float32),
        ],
    )(xr)
    return xt, xs.reshape(M)


def _sc_body(xs_hbm, xt_hbm, feats_hbm, crd_hbm,
             xs_c, order2d, crd_v, cnt_stage, cnt_v,
             row0, row1, row2, row3,
             cnt_sh, gs0, gs1, gs2, gs3, ws0, ws1, ws2, ws3):
    rows = (row0, row1, row2, row3)
    gsems = (gs0, gs1, gs2, gs3)
    wsems = (ws0, ws1, ws2, ws3)

    c = lax.axis_index("c")
    s = lax.axis_index("s")
    w = c * 16 + s
    w2 = (1 - c) * 16 + s
    o = w * CHUNK

    iota = lax.iota(jnp.int32, 16)
    zero16 = jnp.zeros((16,), jnp.int32)

    def _count(chunk_id):
        pltpu.sync_copy(xs_hbm.at[pl.ds(chunk_id * CHUNK, CHUNK)], xs_c)

        def body(i, acc):
            v = xs_c[pl.ds(i * 16, 16)]
            ai = jnp.where(v != 0.0, 1, 0).astype(jnp.int32)
            return acc + jnp.sum(ai, axis=0)

        return lax.fori_loop(0, CHUNK // 16, body, jnp.int32(0))

    n1 = _count(w)
    n2 = _count(w2)

    for kk in range(8):
        cnt_stage[pl.ds(kk * 16, 16)] = jnp.full((16,), n1, jnp.int32)
    pltpu.sync_copy(cnt_stage, cnt_sh.at[w])
    for kk in range(8):
        cnt_stage[pl.ds(kk * 16, 16)] = jnp.full((16,), n2, jnp.int32)
    pltpu.sync_copy(cnt_stage, cnt_sh.at[w2])
    plsc.subcore_barrier()
    pltpu.sync_copy(cnt_sh, cnt_v)

    lo = plsc.load_gather(cnt_v, [iota, zero16])
    hi = plsc.load_gather(cnt_v, [iota + 16, zero16])
    sum_lo = jnp.sum(lo, axis=0)
    tot_a = sum_lo + jnp.sum(hi, axis=0)
    ex_lo = plsc.cumsum(lo) - lo
    ex_hi = plsc.cumsum(hi) - hi + sum_lo

    def _process_chunk(w_, n_w, base_a):
        ib = tot_a + (w_ * CHUNK - base_a)
        ov_a = jnp.logical_and(base_a < o + CHUNK, base_a + n_w > o)
        ov_i = jnp.logical_and(ib < o + CHUNK, ib + (CHUNK - n_w) > o)

        @pl.when(jnp.logical_or(ov_a, ov_i))
        def _():
            pltpu.sync_copy(xs_hbm.at[pl.ds(w_ * CHUNK, CHUNK)], xs_c)

            def stream(i, carry):
                v = xs_c[pl.ds(i * 16, 16)]
                act = v != 0.0
                ai = jnp.where(act, 1, 0).astype(jnp.int32)
                excl = plsc.cumsum(ai) - ai + carry
                li = i * 16 + iota
                out = jnp.where(act, base_a + excl, ib + (li - excl)) - o
                msk = jnp.logical_and(out >= 0, out < CHUNK)
                idx = jnp.minimum(jnp.maximum(out, 0), CHUNK - 1)
                g = w_ * CHUNK + li
                plsc.store_scatter(
                    order2d,
                    [lax.shift_right_logical(idx, 7),
                     jnp.bitwise_and(idx, 127)],
                    g, mask=msk)
                bc = lax.shift_right_logical(g, 14)
                hc = jnp.bitwise_and(lax.shift_right_logical(g, 7), 127)
                wc = jnp.bitwise_and(g, 127)
                idx3 = idx * 3
                plsc.store_scatter(crd_v, [idx3], bc, mask=msk)
                plsc.store_scatter(crd_v, [idx3 + 1], hc, mask=msk)
                plsc.store_scatter(crd_v, [idx3 + 2], wc, mask=msk)
                return carry + jnp.sum(ai, axis=0)

            lax.fori_loop(0, CHUNK // 16, stream, jnp.int32(0))

    for w_ in range(NW):
        if w_ < 16:
            _process_chunk(w_, lo[w_], ex_lo[w_])
        else:
            _process_chunk(w_, hi[w_ - 16], ex_hi[w_ - 16])

    pltpu.sync_copy(crd_v, crd_hbm.at[pl.ds(o * 3, CHUNK * 3)])

    def _g(j, b):
        return pltpu.async_copy(xt_hbm.at[order2d.at[j]], rows[b], gsems[b])

    def _w(j, b):
        return pltpu.async_copy(rows[b],
                                feats_hbm.at[pl.ds(o + j * 128, 128)],
                                wsems[b])

    gd = {}
    wd = {}
    for j in range(4):
        gd[j] = _g(j, j)
    for j in range(SLABS):
        b = j & 3
        gd[j].wait()
        wd[j] = _w(j, b)
        if j + 4 < SLABS:
            wd[j].wait()
            gd[j + 4] = _g(j + 4, b)
    for j in range(SLABS - 4, SLABS):
        wd[j].wait()


def _sc_compact(xs, xt):
    mesh = plsc.VectorSubcoreMesh(core_axis_name="c", subcore_axis_name="s",
                                  num_cores=2, num_subcores=16)
    kern = pl.kernel(
        _sc_body,
        out_type=[
            jax.ShapeDtypeStruct((M, C), jnp.float32),
            jax.ShapeDtypeStruct((M * 3,), jnp.int32),
        ],
        mesh=mesh,
        scratch_types=[
            pltpu.VMEM((CHUNK,), jnp.float32),
            pltpu.VMEM((SLABS, 128), jnp.int32),
            pltpu.VMEM((CHUNK * 3,), jnp.int32),
            pltpu.VMEM((128,), jnp.int32),
            pltpu.VMEM((NW, 128), jnp.int32),
            pltpu.VMEM((128, C), jnp.float32),
            pltpu.VMEM((128, C), jnp.float32),
            pltpu.VMEM((128, C), jnp.float32),
            pltpu.VMEM((128, C), jnp.float32),
            pltpu.VMEM_SHARED((NW, 128), jnp.int32),
            pltpu.SemaphoreType.DMA,
            pltpu.SemaphoreType.DMA,
            pltpu.SemaphoreType.DMA,
            pltpu.SemaphoreType.DMA,
            pltpu.SemaphoreType.DMA,
            pltpu.SemaphoreType.DMA,
            pltpu.SemaphoreType.DMA,
            pltpu.SemaphoreType.DMA,
        ],
        compiler_params=pltpu.CompilerParams(needs_layout_passes=False),
    )
    return kern(xs, xt)


@jax.jit
def kernel(x):
    xt, xs = _tc_transpose(x)
    feats, coords_flat = _sc_compact(xs, xt)
    return coords_flat.reshape(M, 3), feats


# speedup vs baseline: 2.2366x; 1.0017x over previous
import jax
import jax.numpy as jnp
from jax import lax
from jax.experimental import pallas as pl
from jax.experimental.pallas import tpu as pltpu
from jax.experimental.pallas import tpu_sc as plsc

B, C, H, W = 8, 128, 128, 128
HW = H * W
M = B * HW
NBLK = 2
BLK = HW // NBLK
NW = 32
CHUNK = M // NW
SLABS = CHUNK // 128


def _tc_body(x_ref, xt_ref, xs_ref):
    xin = x_ref[0]
    xt_ref[...] = xin.T
    xs_ref[0, 0, :] = jnp.sum(jnp.abs(xin), axis=0)


def _tc_transpose(x):
    xr = x.reshape(B, C, HW)
    xt, xs = pl.pallas_call(
        _tc_body,
        grid=(B, NBLK),
        in_specs=[pl.BlockSpec((1, C, BLK), lambda b, s: (b, 0, s))],
        out_specs=[
            pl.BlockSpec((BLK, C), lambda b, s: (b * NBLK + s, 0)),
            pl.BlockSpec((1, 1, BLK), lambda b, s: (b * NBLK + s, 0, 0)),
        ],
        out_shape=[
            jax.ShapeDtypeStruct((M, C), jnp.float32),
            jax.ShapeDtypeStruct((B * NBLK, 1, BLK), jnp.float32),
        ],
    )(xr)
    return xt, xs.reshape(M)


def _sc_body(xs_hbm, xt_hbm, feats_hbm, crd_hbm,
             xs_c, order2d, crd_v, cnt_stage, cnt_v,
             row0, row1, row2, row3, row4, row5,
             cnt_sh, gs0, gs1, gs2, gs3, gs4, gs5,
             ws0, ws1, ws2, ws3, ws4, ws5):
    rows = (row0, row1, row2, row3, row4, row5)
    gsems = (gs0, gs1, gs2, gs3, gs4, gs5)
    wsems = (ws0, ws1, ws2, ws3, ws4, ws5)

    c = lax.axis_index("c")
    s = lax.axis_index("s")
    w = c * 16 + s
    w2 = (1 - c) * 16 + s
    o = w * CHUNK

    iota = lax.iota(jnp.int32, 16)
    zero16 = jnp.zeros((16,), jnp.int32)

    def _count(chunk_id):
        pltpu.sync_copy(xs_hbm.at[pl.ds(chunk_id * CHUNK, CHUNK)], xs_c)

        def body(i, acc):
            v = xs_c[pl.ds(i * 16, 16)]
            ai = jnp.where(v != 0.0, 1, 0).astype(jnp.int32)
            return acc + jnp.sum(ai, axis=0)

        return lax.fori_loop(0, CHUNK // 16, body, jnp.int32(0))

    n1 = _count(w)
    n2 = _count(w2)

    for kk in range(8):
        cnt_stage[pl.ds(kk * 16, 16)] = jnp.full((16,), n1, jnp.int32)
    pltpu.sync_copy(cnt_stage, cnt_sh.at[w])
    for kk in range(8):
        cnt_stage[pl.ds(kk * 16, 16)] = jnp.full((16,), n2, jnp.int32)
    pltpu.sync_copy(cnt_stage, cnt_sh.at[w2])
    plsc.subcore_barrier()
    pltpu.sync_copy(cnt_sh, cnt_v)

    lo = plsc.load_gather(cnt_v, [iota, zero16])
    hi = plsc.load_gather(cnt_v, [iota + 16, zero16])
    sum_lo = jnp.sum(lo, axis=0)
    tot_a = sum_lo + jnp.sum(hi, axis=0)
    ex_lo = plsc.cumsum(lo) - lo
    ex_hi = plsc.cumsum(hi) - hi + sum_lo

    def _process_chunk(w_, n_w, base_a):
        ib = tot_a + (w_ * CHUNK - base_a)
        ov_a = jnp.logical_and(base_a < o + CHUNK, base_a + n_w > o)
        ov_i = jnp.logical_and(ib < o + CHUNK, ib + (CHUNK - n_w) > o)

        @pl.when(jnp.logical_or(ov_a, ov_i))
        def _():
            pltpu.sync_copy(xs_hbm.at[pl.ds(w_ * CHUNK, CHUNK)], xs_c)

            def stream(i, carry):
                v = xs_c[pl.ds(i * 16, 16)]
                act = v != 0.0
                ai = jnp.where(act, 1, 0).astype(jnp.int32)
                excl = plsc.cumsum(ai) - ai + carry
                li = i * 16 + iota
                out = jnp.where(act, base_a + excl, ib + (li - excl)) - o
                msk = jnp.logical_and(out >= 0, out < CHUNK)
                idx = jnp.minimum(jnp.maximum(out, 0), CHUNK - 1)
                g = w_ * CHUNK + li
                plsc.store_scatter(
                    order2d,
                    [lax.shift_right_logical(idx, 7),
                     jnp.bitwise_and(idx, 127)],
                    g, mask=msk)
                bc = lax.shift_right_logical(g, 14)
                hc = jnp.bitwise_and(lax.shift_right_logical(g, 7), 127)
                wc = jnp.bitwise_and(g, 127)
                idx3 = idx * 3
                plsc.store_scatter(crd_v, [idx3], bc, mask=msk)
                plsc.store_scatter(crd_v, [idx3 + 1], hc, mask=msk)
                plsc.store_scatter(crd_v, [idx3 + 2], wc, mask=msk)
                return carry + jnp.sum(ai, axis=0)

            lax.fori_loop(0, CHUNK // 16, stream, jnp.int32(0))

    for w_ in range(NW):
        if w_ < 16:
            _process_chunk(w_, lo[w_], ex_lo[w_])
        else:
            _process_chunk(w_, hi[w_ - 16], ex_hi[w_ - 16])

    pltpu.sync_copy(crd_v, crd_hbm.at[pl.ds(o * 3, CHUNK * 3)])

    def _g(j, b):
        return pltpu.async_copy(xt_hbm.at[order2d.at[j]], rows[b], gsems[b])

    def _w(j, b):
        return pltpu.async_copy(rows[b],
                                feats_hbm.at[pl.ds(o + j * 128, 128)],
                                wsems[b])

    NB = 6
    gd = {}
    wd = {}
    for j in range(NB):
        gd[j] = _g(j, j)
    for j in range(SLABS):
        b = j % NB
        gd[j].wait()
        wd[j] = _w(j, b)
        if j + NB < SLABS:
            wd[j].wait()
            gd[j + NB] = _g(j + NB, b)
    for j in range(SLABS - NB, SLABS):
        wd[j].wait()


def _sc_compact(xs, xt):
    mesh = plsc.VectorSubcoreMesh(core_axis_name="c", subcore_axis_name="s",
                                  num_cores=2, num_subcores=16)
    kern = pl.kernel(
        _sc_body,
        out_type=[
            jax.ShapeDtypeStruct((M, C), jnp.float32),
            jax.ShapeDtypeStruct((M * 3,), jnp.int32),
        ],
        mesh=mesh,
        scratch_types=[
            pltpu.VMEM((CHUNK,), jnp.float32),
            pltpu.VMEM((SLABS, 128), jnp.int32),
            pltpu.VMEM((CHUNK * 3,), jnp.int32),
            pltpu.VMEM((128,), jnp.int32),
            pltpu.VMEM((NW, 128), jnp.int32),
            pltpu.VMEM((128, C), jnp.float32),
            pltpu.VMEM((128, C), jnp.float32),
            pltpu.VMEM((128, C), jnp.float32),
            pltpu.VMEM((128, C), jnp.float32),
            pltpu.VMEM((128, C), jnp.float32),
            pltpu.VMEM((128, C), jnp.float32),
            pltpu.VMEM_SHARED((NW, 128), jnp.int32),
        ] + [pltpu.SemaphoreType.DMA] * 12,
        compiler_params=pltpu.CompilerParams(needs_layout_passes=False),
    )
    return kern(xs, xt)


@jax.jit
def kernel(x):
    xt, xs = _tc_transpose(x)
    feats, coords_flat = _sc_compact(xs, xt)
    return coords_flat.reshape(M, 3), feats


# speedup vs baseline: 2.2442x; 1.0034x over previous
import jax
import jax.numpy as jnp
from jax import lax
from jax.experimental import pallas as pl
from jax.experimental.pallas import tpu as pltpu
from jax.experimental.pallas import tpu_sc as plsc

B, C, H, W = 8, 128, 128, 128
HW = H * W
M = B * HW
NBLK = 1
BLK = HW // NBLK
NW = 32
CHUNK = M // NW
SLABS = CHUNK // 128


def _tc_body(x_ref, xt_ref, xs_ref):
    xin = x_ref[0]
    xt_ref[...] = xin.T
    xs_ref[0, 0, :] = jnp.sum(jnp.abs(xin), axis=0)


def _tc_transpose(x):
    xr = x.reshape(B, C, HW)
    xt, xs = pl.pallas_call(
        _tc_body,
        grid=(B, NBLK),
        in_specs=[pl.BlockSpec((1, C, BLK), lambda b, s: (b, 0, s))],
        out_specs=[
            pl.BlockSpec((BLK, C), lambda b, s: (b * NBLK + s, 0)),
            pl.BlockSpec((1, 1, BLK), lambda b, s: (b * NBLK + s, 0, 0)),
        ],
        out_shape=[
            jax.ShapeDtypeStruct((M, C), jnp.float32),
            jax.ShapeDtypeStruct((B * NBLK, 1, BLK), jnp.float32),
        ],
        compiler_params=pltpu.CompilerParams(
            vmem_limit_bytes=110 * 1024 * 1024),
    )(xr)
    return xt, xs.reshape(M)


def _sc_body(xs_hbm, xt_hbm, feats_hbm, crd_hbm,
             xs_c, order2d, crd_v, cnt_stage, cnt_v,
             row0, row1, row2, row3, row4, row5,
             cnt_sh, gs0, gs1, gs2, gs3, gs4, gs5,
             ws0, ws1, ws2, ws3, ws4, ws5):
    rows = (row0, row1, row2, row3, row4, row5)
    gsems = (gs0, gs1, gs2, gs3, gs4, gs5)
    wsems = (ws0, ws1, ws2, ws3, ws4, ws5)

    c = lax.axis_index("c")
    s = lax.axis_index("s")
    w = c * 16 + s
    w2 = (1 - c) * 16 + s
    o = w * CHUNK

    iota = lax.iota(jnp.int32, 16)
    zero16 = jnp.zeros((16,), jnp.int32)

    def _count(chunk_id):
        pltpu.sync_copy(xs_hbm.at[pl.ds(chunk_id * CHUNK, CHUNK)], xs_c)

        def body(i, acc):
            v = xs_c[pl.ds(i * 16, 16)]
            ai = jnp.where(v != 0.0, 1, 0).astype(jnp.int32)
            return acc + jnp.sum(ai, axis=0)

        return lax.fori_loop(0, CHUNK // 16, body, jnp.int32(0))

    n1 = _count(w)
    n2 = _count(w2)

    for kk in range(8):
        cnt_stage[pl.ds(kk * 16, 16)] = jnp.full((16,), n1, jnp.int32)
    pltpu.sync_copy(cnt_stage, cnt_sh.at[w])
    for kk in range(8):
        cnt_stage[pl.ds(kk * 16, 16)] = jnp.full((16,), n2, jnp.int32)
    pltpu.sync_copy(cnt_stage, cnt_sh.at[w2])
    plsc.subcore_barrier()
    pltpu.sync_copy(cnt_sh, cnt_v)

    lo = plsc.load_gather(cnt_v, [iota, zero16])
    hi = plsc.load_gather(cnt_v, [iota + 16, zero16])
    sum_lo = jnp.sum(lo, axis=0)
    tot_a = sum_lo + jnp.sum(hi, axis=0)
    ex_lo = plsc.cumsum(lo) - lo
    ex_hi = plsc.cumsum(hi) - hi + sum_lo

    def _process_chunk(w_, n_w, base_a):
        ib = tot_a + (w_ * CHUNK - base_a)
        ov_a = jnp.logical_and(base_a < o + CHUNK, base_a + n_w > o)
        ov_i = jnp.logical_and(ib < o + CHUNK, ib + (CHUNK - n_w) > o)

        @pl.when(jnp.logical_or(ov_a, ov_i))
        def _():
            pltpu.sync_copy(xs_hbm.at[pl.ds(w_ * CHUNK, CHUNK)], xs_c)

            def stream(i, carry):
                v = xs_c[pl.ds(i * 16, 16)]
                act = v != 0.0
                ai = jnp.where(act, 1, 0).astype(jnp.int32)
                excl = plsc.cumsum(ai) - ai + carry
                li = i * 16 + iota
                out = jnp.where(act, base_a + excl, ib + (li - excl)) - o
                msk = jnp.logical_and(out >= 0, out < CHUNK)
                idx = jnp.minimum(jnp.maximum(out, 0), CHUNK - 1)
                g = w_ * CHUNK + li
                plsc.store_scatter(
                    order2d,
                    [lax.shift_right_logical(idx, 7),
                     jnp.bitwise_and(idx, 127)],
                    g, mask=msk)
                bc = lax.shift_right_logical(g, 14)
                hc = jnp.bitwise_and(lax.shift_right_logical(g, 7), 127)
                wc = jnp.bitwise_and(g, 127)
                idx3 = idx * 3
                plsc.store_scatter(crd_v, [idx3], bc, mask=msk)
                plsc.store_scatter(crd_v, [idx3 + 1], hc, mask=msk)
                plsc.store_scatter(crd_v, [idx3 + 2], wc, mask=msk)
                return carry + jnp.sum(ai, axis=0)

            lax.fori_loop(0, CHUNK // 16, stream, jnp.int32(0))

    for w_ in range(NW):
        if w_ < 16:
            _process_chunk(w_, lo[w_], ex_lo[w_])
        else:
            _process_chunk(w_, hi[w_ - 16], ex_hi[w_ - 16])

    pltpu.sync_copy(crd_v, crd_hbm.at[pl.ds(o * 3, CHUNK * 3)])

    def _g(j, b):
        return pltpu.async_copy(xt_hbm.at[order2d.at[j]], rows[b], gsems[b])

    def _w(j, b):
        return pltpu.async_copy(rows[b],
                                feats_hbm.at[pl.ds(o + j * 128, 128)],
                                wsems[b])

    NB = 6
    gd = {}
    wd = {}
    for j in range(NB):
        gd[j] = _g(j, j)
    for j in range(SLABS):
        b = j % NB
        gd[j].wait()
        wd[j] = _w(j, b)
        if j + NB < SLABS:
            wd[j].wait()
            gd[j + NB] = _g(j + NB, b)
    for j in range(SLABS - NB, SLABS):
        wd[j].wait()


def _sc_compact(xs, xt):
    mesh = plsc.VectorSubcoreMesh(core_axis_name="c", subcore_axis_name="s",
                                  num_cores=2, num_subcores=16)
    kern = pl.kernel(
        _sc_body,
        out_type=[
            jax.ShapeDtypeStruct((M, C), jnp.float32),
            jax.ShapeDtypeStruct((M * 3,), jnp.int32),
        ],
        mesh=mesh,
        scratch_types=[
            pltpu.VMEM((CHUNK,), jnp.float32),
            pltpu.VMEM((SLABS, 128), jnp.int32),
            pltpu.VMEM((CHUNK * 3,), jnp.int32),
            pltpu.VMEM((128,), jnp.int32),
            pltpu.VMEM((NW, 128), jnp.int32),
            pltpu.VMEM((128, C), jnp.float32),
            pltpu.VMEM((128, C), jnp.float32),
            pltpu.VMEM((128, C), jnp.float32),
            pltpu.VMEM((128, C), jnp.float32),
            pltpu.VMEM((128, C), jnp.float32),
            pltpu.VMEM((128, C), jnp.float32),
            pltpu.VMEM_SHARED((NW, 128), jnp.int32),
        ] + [pltpu.SemaphoreType.DMA] * 12,
        compiler_params=pltpu.CompilerParams(needs_layout_passes=False),
    )
    return kern(xs, xt)


@jax.jit
def kernel(x):
    xt, xs = _tc_transpose(x)
    feats, coords_flat = _sc_compact(xs, xt)
    return coords_flat.reshape(M, 3), feats
